# baseline (speedup 1.0000x reference)
import jax
import jax.numpy as jnp
from jax import lax
from jax.experimental import pallas as pl
from jax.experimental.pallas import tpu as pltpu
from jax.experimental.pallas import tpu_sc as plsc

N = 10000
E = 320000
D = 128
NUM_CORES = 2
NUM_SUBCORES = 16
EDGES_PER_TILE = E // NUM_SUBCORES
CHUNK = 128
FULL_CHUNKS = EDGES_PER_TILE // CHUNK
REM = EDGES_PER_TILE - FULL_CHUNKS * CHUNK
NROWS = 2
NIDX = 4
NPAD = 10240
STRIPE = NPAD // NUM_SUBCORES
C_S2D = 0.5
C_D2S = 0.5


def _sc_body(x_hbm, edge_hbm, s_out, deg_out,
             gbig0, gbig1, sidx0, sidx1, sidx2, sidx3,
             rows0, rows1, ones_v, dstage_v, gidx_r, sidx_r,
             accum, degacc,
             gs0, gs1, is0, is1, is2, is3, gb0, gb1, ss0, ss1, ss2, ss3):
    c = lax.axis_index("c")
    s = lax.axis_index("s")
    g_base = c * E
    s_base = (1 - c) * E

    zero16 = jnp.zeros((16,), jnp.float32)
    one16 = jnp.ones((16,), jnp.float32)

    def _zero_row(i, carry):
        for j in range(D // 16):
            rows0[i, pl.ds(j * 16, 16)] = zero16
        return carry

    def _zero_dstage(i, carry):
        dstage_v[pl.ds(i * 16, 16)] = zero16
        return carry

    def _fill_ones(i, carry):
        ones_v[pl.ds(i * 16, 16)] = one16
        return carry

    lax.fori_loop(0, CHUNK, _zero_row, 0)
    lax.fori_loop(0, STRIPE // 16, _zero_dstage, 0)
    lax.fori_loop(0, CHUNK // 16, _fill_ones, 0)

    r0 = pl.multiple_of(s * STRIPE, 8)
    for k in range(STRIPE // CHUNK):
        pltpu.sync_copy(rows0, accum.at[pl.ds(r0 + k * CHUNK, CHUNK)])
    _ztail = STRIPE - (STRIPE // CHUNK) * CHUNK
    if _ztail:
        pltpu.sync_copy(rows0.at[pl.ds(0, _ztail)],
                        accum.at[pl.ds(r0 + STRIPE - _ztail, _ztail)])
    pltpu.sync_copy(dstage_v, degacc.at[pl.ds(r0, STRIPE)])
    plsc.subcore_barrier()

    base = s * EDGES_PER_TILE
    sidx_bufs = (sidx0, sidx1, sidx2, sidx3)
    gbig_bufs = (gbig0, gbig1)
    rows_bufs = (rows0, rows1)
    gsems = (gs0, gs1)
    isems = (is0, is1, is2, is3)
    gbsems = (gb0, gb1)
    ssems = (ss0, ss1, ss2, ss3)

    def _g_slice(g):
        return edge_hbm.at[pl.ds(pl.multiple_of(g_base + base + g * CHUNK, 8),
                                 CHUNK)]

    def _s_slice(g):
        return edge_hbm.at[pl.ds(pl.multiple_of(s_base + base + g * CHUNK, 8),
                                 CHUNK)]

    def _gb_slice(q):
        return edge_hbm.at[pl.ds(
            pl.multiple_of(g_base + base + q * (4 * CHUNK), 8), 4 * CHUNK)]

    def _gbig_start(g_first, qp):
        pltpu.async_copy(
            edge_hbm.at[pl.ds(pl.multiple_of(
                g_base + base + g_first * CHUNK, 8), 4 * CHUNK)],
            gbig_bufs[qp % 2], gbsems[qp % 2])

    def _gbig_wait(qp):
        pltpu.make_async_copy(_gb_slice(0), gbig_bufs[qp % 2],
                              gbsems[qp % 2]).wait()

    def _gidx_of(h):
        return gbig_bufs[(h // 4) % 2].at[pl.ds((h % 4) * CHUNK, CHUNK)]

    def _idx_start(g, h):
        sidx = sidx_bufs[h % NIDX]
        pltpu.async_copy(_s_slice(g), sidx, isems[h % NIDX])

    def _idx_wait(h):
        sidx = sidx_bufs[h % NIDX]
        pltpu.make_async_copy(_s_slice(0), sidx, isems[h % NIDX]).wait()

    def _gather_start(h):
        pltpu.async_copy(x_hbm.at[_gidx_of(h)], rows_bufs[h % NROWS],
                         gsems[h % NROWS])

    def _gather_wait(h):
        pltpu.make_async_copy(x_hbm.at[_gidx_of(h)], rows_bufs[h % NROWS],
                              gsems[h % NROWS]).wait()

    def _scatter_start(h):
        sidx = sidx_bufs[h % NIDX]
        ssem = ssems[h % NIDX]
        pltpu.async_copy(rows_bufs[h % NROWS], accum.at[sidx], ssem, add=True)
        pltpu.async_copy(ones_v, degacc.at[sidx], ssem, add=True)

    def _scatter_wait(h):
        sidx = sidx_bufs[h % NIDX]
        ssem = ssems[h % NIDX]
        pltpu.make_async_copy(rows_bufs[h % NROWS], accum.at[sidx],
                              ssem).wait()
        pltpu.make_async_copy(ones_v, degacc.at[sidx], ssem).wait()

    pltpu.sync_copy(_gb_slice(0), gbig0)
    pltpu.sync_copy(_s_slice(0), sidx0)
    _gbig_start(4, 1)
    _gather_start(0)
    _idx_start(1, 1)

    def _step(g, h, warm, gb_start=True):
        _gather_wait(h)
        if warm:
            _scatter_wait(h - 1)
        if h % 4 == 3:
            _gbig_wait(h // 4 + 1)
            if gb_start:
                _gbig_start(g + 5, h // 4 + 2)
        _idx_wait(h + 1)
        _gather_start(h + 1)
        _scatter_start(h)
        _idx_start(g + 2, h + 2)

    _step(0, 0, False)
    _step(1, 1, True)

    def _oct(i, carry):
        g = 8 * i + 2
        for p in range(8):
            _step(g + p, 2 + p, True)
        return carry

    n_loops = (FULL_CHUNKS - 2 - 10) // 8
    lax.fori_loop(0, n_loops, _oct, 0)
    g_pe = 2 + 8 * n_loops
    for p in range(FULL_CHUNKS - 2 - g_pe):
        h = g_pe + p
        _step(h, h, True, gb_start=(h <= 147))
    hA, hB = FULL_CHUNKS - 2, FULL_CHUNKS - 1
    _gather_wait(hA)
    _scatter_wait(hA - 1)
    _idx_wait(hB)
    _gather_start(hB)
    _scatter_start(hA)
    _gather_wait(hB)
    _scatter_wait(hA)
    _scatter_start(hB)
    _scatter_wait(hB)

    offr = base + FULL_CHUNKS * CHUNK
    pltpu.sync_copy(edge_hbm.at[pl.ds(pl.multiple_of(g_base + offr, 8), REM)],
                    gidx_r)
    pltpu.sync_copy(edge_hbm.at[pl.ds(pl.multiple_of(s_base + offr, 8), REM)],
                    sidx_r)
    pltpu.async_copy(x_hbm.at[gidx_r], rows0.at[pl.ds(0, REM)], gs0).wait()
    pltpu.sync_copy(rows0.at[pl.ds(0, REM)], accum.at[sidx_r], add=True)
    pltpu.sync_copy(ones_v.at[pl.ds(0, REM)], degacc.at[sidx_r], add=True)

    plsc.subcore_barrier()

    pltpu.sync_copy(accum.at[pl.ds(r0, STRIPE)],
                    s_out.at[c, pl.ds(r0, STRIPE)])
    pltpu.sync_copy(degacc.at[pl.ds(r0, STRIPE)],
                    deg_out.at[pl.ds(pl.multiple_of(c * NPAD + r0, 8),
                                     STRIPE)])


_sc_aggregate = pl.kernel(
    _sc_body,
    out_type=(
        jax.ShapeDtypeStruct((NUM_CORES, NPAD, D), jnp.float32),
        jax.ShapeDtypeStruct((NUM_CORES * NPAD,), jnp.float32),
    ),
    mesh=plsc.VectorSubcoreMesh(
        core_axis_name="c", subcore_axis_name="s",
        num_cores=NUM_CORES, num_subcores=NUM_SUBCORES),
    scratch_types=(
        [pltpu.VMEM((4 * CHUNK,), jnp.int32) for _ in range(2)]
        + [pltpu.VMEM((CHUNK,), jnp.int32) for _ in range(4)]
        + [pltpu.VMEM((CHUNK, D), jnp.float32) for _ in range(2)]
        + [
            pltpu.VMEM((CHUNK,), jnp.float32),
            pltpu.VMEM((STRIPE,), jnp.float32),
            pltpu.VMEM((REM,), jnp.int32),
            pltpu.VMEM((REM,), jnp.int32),
            pltpu.VMEM_SHARED((NPAD, D), jnp.float32),
            pltpu.VMEM_SHARED((NPAD,), jnp.float32),
        ]
        + [pltpu.SemaphoreType.DMA] * 12
    ),
)


BLK = 1000


def _fin_body(x_ref, s0_ref, s1_ref, d0_ref, d1_ref, ws_ref, w1_ref, w2_ref,
              bs_ref, b1_ref, b2_ref, o_ref):
    inv0 = C_S2D / jnp.maximum(d0_ref[...], 1.0)
    inv1 = C_D2S / jnp.maximum(d1_ref[...], 1.0)
    acc = jnp.dot(x_ref[...], ws_ref[...], preferred_element_type=jnp.float32)
    acc = acc + jnp.dot(s0_ref[...] * inv0, w1_ref[...],
                        preferred_element_type=jnp.float32)
    acc = acc + jnp.dot(s1_ref[...] * inv1, w2_ref[...],
                        preferred_element_type=jnp.float32)
    bias = bs_ref[...] + C_S2D * b1_ref[...] + C_D2S * b2_ref[...]
    o_ref[...] = acc + bias[None, :]


def _finalize(x, s0, s1, d0, d1, w_self, w_s2d, w_d2s, b_self, b_s2d, b_d2s):
    row_spec = pl.BlockSpec((BLK, D), lambda i: (i, 0))
    deg_spec = pl.BlockSpec((BLK, 1), lambda i: (i, 0))
    w_spec = pl.BlockSpec((D, D), lambda i: (0, 0))
    b_spec = pl.BlockSpec((D,), lambda i: (0,))
    return pl.pallas_call(
        _fin_body,
        grid=(N // BLK,),
        in_specs=[row_spec, row_spec, row_spec, deg_spec, deg_spec,
                  w_spec, w_spec, w_spec, b_spec, b_spec, b_spec],
        out_specs=row_spec,
        out_shape=jax.ShapeDtypeStruct((N, D), jnp.float32),
    )(x, s0, s1, d0, d1, w_self, w_s2d, w_d2s, b_self, b_s2d, b_d2s)


def kernel(x, edge_index, W_self, b_self, W_s2d, b_s2d, W_d2s, b_d2s):
    edge_flat = edge_index.reshape(2 * E)
    sums, degs = _sc_aggregate(x, edge_flat)
    d2 = degs.reshape(NUM_CORES, NPAD)
    return _finalize(x, sums[0], sums[1],
                     d2[0, :N].reshape(N, 1), d2[1, :N].reshape(N, 1),
                     W_self, W_s2d, W_d2s, b_self, b_s2d, b_d2s)


# speedup vs baseline: 1.0068x; 1.0068x over previous
import jax
import jax.numpy as jnp
from jax import lax
from jax.experimental import pallas as pl
from jax.experimental.pallas import tpu as pltpu
from jax.experimental.pallas import tpu_sc as plsc

N = 10000
E = 320000
D = 128
NUM_CORES = 2
NUM_SUBCORES = 16
EDGES_PER_TILE = E // NUM_SUBCORES
CHUNK = 128
FULL_CHUNKS = EDGES_PER_TILE // CHUNK
REM = EDGES_PER_TILE - FULL_CHUNKS * CHUNK
NROWS = 2
NIDX = 4
NPAD = 10240
STRIPE = NPAD // NUM_SUBCORES
C_S2D = 0.5
C_D2S = 0.5


def _sc_body(x_hbm, edge_hbm, s_out, deg_out,
             gidx0, sidx0, gidx1, sidx1, gidx2, sidx2, gidx3, sidx3,
             rows0, rows1, ones_v, dstage_v, gidx_r, sidx_r,
             accum, degacc,
             gs0, gs1, is0, is1, is2, is3, ss0, ss1, ss2, ss3):
    c = lax.axis_index("c")
    s = lax.axis_index("s")
    g_base = c * E
    s_base = (1 - c) * E

    zero16 = jnp.zeros((16,), jnp.float32)
    one16 = jnp.ones((16,), jnp.float32)

    def _zero_row(i, carry):
        for j in range(D // 16):
            rows0[i, pl.ds(j * 16, 16)] = zero16
        return carry

    def _zero_dstage(i, carry):
        dstage_v[pl.ds(i * 16, 16)] = zero16
        return carry

    def _fill_ones(i, carry):
        ones_v[pl.ds(i * 16, 16)] = one16
        return carry

    lax.fori_loop(0, CHUNK, _zero_row, 0)
    lax.fori_loop(0, STRIPE // 16, _zero_dstage, 0)
    lax.fori_loop(0, CHUNK // 16, _fill_ones, 0)

    r0 = pl.multiple_of(s * STRIPE, 8)
    for k in range(STRIPE // CHUNK):
        pltpu.sync_copy(rows0, accum.at[pl.ds(r0 + k * CHUNK, CHUNK)])
    _ztail = STRIPE - (STRIPE // CHUNK) * CHUNK
    if _ztail:
        pltpu.sync_copy(rows0.at[pl.ds(0, _ztail)],
                        accum.at[pl.ds(r0 + STRIPE - _ztail, _ztail)])
    pltpu.sync_copy(dstage_v, degacc.at[pl.ds(r0, STRIPE)])
    plsc.subcore_barrier()

    base = s * EDGES_PER_TILE
    idx_bufs = ((gidx0, sidx0), (gidx1, sidx1), (gidx2, sidx2),
                (gidx3, sidx3))
    rows_bufs = (rows0, rows1)
    gsems = (gs0, gs1)
    isems = (is0, is1, is2, is3)
    ssems = (ss0, ss1, ss2, ss3)

    def _g_slice(g):
        return edge_hbm.at[pl.ds(pl.multiple_of(g_base + base + g * CHUNK, 8),
                                 CHUNK)]

    def _s_slice(g):
        return edge_hbm.at[pl.ds(pl.multiple_of(s_base + base + g * CHUNK, 8),
                                 CHUNK)]

    def _idx_start(g, h):
        gidx, sidx = idx_bufs[h % NIDX]
        isem = isems[h % NIDX]
        pltpu.async_copy(_g_slice(g), gidx, isem)
        pltpu.async_copy(_s_slice(g), sidx, isem)

    def _idx_wait(h):
        gidx, sidx = idx_bufs[h % NIDX]
        isem = isems[h % NIDX]
        pltpu.make_async_copy(_g_slice(0), gidx, isem).wait()
        pltpu.make_async_copy(_s_slice(0), sidx, isem).wait()

    def _gather_start(h):
        gidx, _ = idx_bufs[h % NIDX]
        pltpu.async_copy(x_hbm.at[gidx], rows_bufs[h % NROWS],
                         gsems[h % NROWS])

    def _gather_wait(h):
        gidx, _ = idx_bufs[h % NIDX]
        pltpu.make_async_copy(x_hbm.at[gidx], rows_bufs[h % NROWS],
                              gsems[h % NROWS]).wait()

    def _scatter_start(h):
        _, sidx = idx_bufs[h % NIDX]
        ssem = ssems[h % NIDX]
        pltpu.async_copy(rows_bufs[h % NROWS], accum.at[sidx], ssem, add=True)
        pltpu.async_copy(ones_v, degacc.at[sidx], ssem, add=True)

    def _scatter_wait(h):
        _, sidx = idx_bufs[h % NIDX]
        ssem = ssems[h % NIDX]
        pltpu.make_async_copy(rows_bufs[h % NROWS], accum.at[sidx],
                              ssem).wait()
        pltpu.make_async_copy(ones_v, degacc.at[sidx], ssem).wait()

    pltpu.sync_copy(_g_slice(0), gidx0)
    pltpu.sync_copy(_s_slice(0), sidx0)
    _gather_start(0)
    _idx_start(1, 1)

    def _step(g, h, warm):
        _gather_wait(h)
        if warm:
            _scatter_wait(h - 1)
        _idx_wait(h + 1)
        _gather_start(h + 1)
        _scatter_start(h)
        _idx_start(g + 2, h + 2)

    _step(0, 0, False)
    _step(1, 1, True)

    def _quad(i, carry):
        g = 4 * i + 2
        for p in range(4):
            _step(g + p, 2 + p, True)
        return carry

    n_loops = (FULL_CHUNKS - 2 - 2) // 4
    lax.fori_loop(0, n_loops, _quad, 0)
    g_pe = 2 + 4 * n_loops
    for p in range(FULL_CHUNKS - 2 - g_pe):
        _step(g_pe + p, g_pe + p, True)
    hA, hB = FULL_CHUNKS - 2, FULL_CHUNKS - 1
    _gather_wait(hA)
    _scatter_wait(hA - 1)
    _idx_wait(hB)
    _gather_start(hB)
    _scatter_start(hA)
    _gather_wait(hB)
    _scatter_wait(hA)
    _scatter_start(hB)
    _scatter_wait(hB)

    offr = base + FULL_CHUNKS * CHUNK
    pltpu.sync_copy(edge_hbm.at[pl.ds(pl.multiple_of(g_base + offr, 8), REM)],
                    gidx_r)
    pltpu.sync_copy(edge_hbm.at[pl.ds(pl.multiple_of(s_base + offr, 8), REM)],
                    sidx_r)
    pltpu.async_copy(x_hbm.at[gidx_r], rows0.at[pl.ds(0, REM)], gs0).wait()
    pltpu.sync_copy(rows0.at[pl.ds(0, REM)], accum.at[sidx_r], add=True)
    pltpu.sync_copy(ones_v.at[pl.ds(0, REM)], degacc.at[sidx_r], add=True)

    plsc.subcore_barrier()

    pltpu.sync_copy(accum.at[pl.ds(r0, STRIPE)],
                    s_out.at[c, pl.ds(r0, STRIPE)])
    pltpu.sync_copy(degacc.at[pl.ds(r0, STRIPE)],
                    deg_out.at[pl.ds(pl.multiple_of(c * NPAD + r0, 8),
                                     STRIPE)])


_sc_aggregate = pl.kernel(
    _sc_body,
    out_type=(
        jax.ShapeDtypeStruct((NUM_CORES, NPAD, D), jnp.float32),
        jax.ShapeDtypeStruct((NUM_CORES * NPAD,), jnp.float32),
    ),
    mesh=plsc.VectorSubcoreMesh(
        core_axis_name="c", subcore_axis_name="s",
        num_cores=NUM_CORES, num_subcores=NUM_SUBCORES),
    scratch_types=(
        [pltpu.VMEM((CHUNK,), jnp.int32) for _ in range(8)]
        + [pltpu.VMEM((CHUNK, D), jnp.float32) for _ in range(2)]
        + [
            pltpu.VMEM((CHUNK,), jnp.float32),
            pltpu.VMEM((STRIPE,), jnp.float32),
            pltpu.VMEM((REM,), jnp.int32),
            pltpu.VMEM((REM,), jnp.int32),
            pltpu.VMEM_SHARED((NPAD, D), jnp.float32),
            pltpu.VMEM_SHARED((NPAD,), jnp.float32),
        ]
        + [pltpu.SemaphoreType.DMA] * 10
    ),
)


BLK = 1000


def _self_body(x_ref, ws_ref, bs_ref, b1_ref, b2_ref, o_ref):
    acc = jnp.dot(x_ref[...], ws_ref[...], preferred_element_type=jnp.float32)
    bias = bs_ref[...] + C_S2D * b1_ref[...] + C_D2S * b2_ref[...]
    o_ref[...] = acc + bias[None, :]


def _fin_body(base_ref, s0_ref, s1_ref, d0_ref, d1_ref, w1_ref, w2_ref,
              o_ref):
    inv0 = C_S2D / jnp.maximum(d0_ref[...], 1.0)
    inv1 = C_D2S / jnp.maximum(d1_ref[...], 1.0)
    acc = base_ref[...]
    acc = acc + jnp.dot(s0_ref[...] * inv0, w1_ref[...],
                        preferred_element_type=jnp.float32)
    acc = acc + jnp.dot(s1_ref[...] * inv1, w2_ref[...],
                        preferred_element_type=jnp.float32)
    o_ref[...] = acc


def _row_spec():
    return pl.BlockSpec((BLK, D), lambda i: (i, 0))


def _self_part(x, w_self, b_self, b_s2d, b_d2s):
    w_spec = pl.BlockSpec((D, D), lambda i: (0, 0))
    b_spec = pl.BlockSpec((D,), lambda i: (0,))
    return pl.pallas_call(
        _self_body,
        grid=(N // BLK,),
        in_specs=[_row_spec(), w_spec, b_spec, b_spec, b_spec],
        out_specs=_row_spec(),
        out_shape=jax.ShapeDtypeStruct((N, D), jnp.float32),
    )(x, w_self, b_self, b_s2d, b_d2s)


def _finalize(base, s0, s1, d0, d1, w_s2d, w_d2s):
    deg_spec = pl.BlockSpec((BLK, 1), lambda i: (i, 0))
    w_spec = pl.BlockSpec((D, D), lambda i: (0, 0))
    return pl.pallas_call(
        _fin_body,
        grid=(N // BLK,),
        in_specs=[_row_spec(), _row_spec(), _row_spec(), deg_spec, deg_spec,
                  w_spec, w_spec],
        out_specs=_row_spec(),
        out_shape=jax.ShapeDtypeStruct((N, D), jnp.float32),
    )(base, s0, s1, d0, d1, w_s2d, w_d2s)


def kernel(x, edge_index, W_self, b_self, W_s2d, b_s2d, W_d2s, b_d2s):
    edge_flat = edge_index.reshape(2 * E)
    base = _self_part(x, W_self, b_self, b_s2d, b_d2s)
    sums, degs = _sc_aggregate(x, edge_flat)
    d2 = degs.reshape(NUM_CORES, NPAD)
    return _finalize(base, sums[0], sums[1],
                     d2[0, :N].reshape(N, 1), d2[1, :N].reshape(N, 1),
                     W_s2d, W_d2s)
